# baseline (speedup 1.0000x reference)
import functools
import math

import jax
import jax.numpy as jnp
from jax import lax
from jax.experimental import pallas as pl
from jax.experimental.pallas import tpu as pltpu
from jax.experimental.pallas import tpu_sc as plsc

D_MODEL = 64
D_PAD = 128
S_LEN = 200
S_PAD = 256
SCALE = math.sqrt(D_MODEL)
LANES = 16

_INFO = plsc.get_sparse_core_info()
_NC = _INFO.num_cores
_NS = _INFO.num_subcores
_NW = _NC * _NS


@functools.lru_cache(maxsize=None)
def _build_gather(n_b: int, vocab: int):
    assert n_b % _NW == 0
    n_chunks = n_b // _NW
    assert n_chunks % 2 == 0
    half = n_chunks // 2

    mesh = plsc.VectorSubcoreMesh(core_axis_name="c", subcore_axis_name="s")

    @functools.partial(
        pl.kernel,
        mesh=mesh,
        out_type=jax.ShapeDtypeStruct((n_b, S_LEN, D_MODEL), jnp.float32),
        scratch_types=[
            pltpu.VMEM((2, S_PAD), jnp.int32),
            pltpu.VMEM((S_LEN, D_PAD), jnp.float32),
            pltpu.VMEM((S_LEN, D_PAD), jnp.float32),
            pltpu.VMEM((S_LEN, D_MODEL), jnp.float32),
            pltpu.VMEM((S_LEN, D_MODEL), jnp.float32),
            pltpu.SemaphoreType.DMA,
            pltpu.SemaphoreType.DMA,
            pltpu.SemaphoreType.DMA,
            pltpu.SemaphoreType.DMA,
        ],
    )
    def gather_kernel(idx_hbm, table_hbm, out_hbm,
                      idx_v, gbuf_a, gbuf_b, obuf_a, obuf_b,
                      sga, sgb, swa, swb):
        wid = lax.axis_index("s") * _NC + lax.axis_index("c")
        row0 = wid * n_chunks

        def fire_gather(g, slot, gbuf, sem):
            pltpu.sync_copy(idx_hbm.at[pl.ds(row0 + g, 1)],
                            idx_v.at[pl.ds(slot, 1)])
            pltpu.async_copy(
                table_hbm.at[idx_v.at[slot, pl.ds(0, 128)]],
                gbuf.at[pl.ds(0, 128), :],
                sem,
            )
            pltpu.async_copy(
                table_hbm.at[idx_v.at[slot, pl.ds(128, S_LEN - 128)]],
                gbuf.at[pl.ds(128, S_LEN - 128), :],
                sem,
            )

        def wait_gather(gbuf, sem):
            pltpu.make_async_copy(
                table_hbm.at[pl.ds(0, S_LEN)], gbuf, sem).wait()

        def wait_write(obuf, sem):
            pltpu.make_async_copy(
                table_hbm.at[pl.ds(0, S_LEN)],
                obuf.at[:, pl.ds(0, D_MODEL)], sem).wait()

        def scale(gbuf, obuf):
            def body(r, carry):
                for c in range(D_MODEL // LANES):
                    v = gbuf[r, pl.ds(c * LANES, LANES)]
                    obuf[r, pl.ds(c * LANES, LANES)] = v * SCALE
                return carry
            lax.fori_loop(0, S_LEN, body, 0, unroll=2)

        def fire_write(g, obuf, sem):
            pltpu.async_copy(obuf, out_hbm.at[row0 + g], sem)

        fire_gather(0, 0, gbuf_a, sga)

        def loop_body(t, carry):
            ga = 2 * t
            gb = 2 * t + 1
            fire_gather(gb, 1, gbuf_b, sgb)
            wait_gather(gbuf_a, sga)

            @pl.when(t > 0)
            def _():
                wait_write(obuf_a, swa)

            scale(gbuf_a, obuf_a)
            fire_write(ga, obuf_a, swa)

            @pl.when(t < half - 1)
            def _():
                fire_gather(ga + 2, 0, gbuf_a, sga)

            wait_gather(gbuf_b, sgb)

            @pl.when(t > 0)
            def _():
                wait_write(obuf_b, swb)

            scale(gbuf_b, obuf_b)
            fire_write(gb, obuf_b, swb)
            return carry

        lax.fori_loop(0, half, loop_body, 0)

        wait_write(obuf_a, swa)
        wait_write(obuf_b, swb)

    return gather_kernel


def kernel(x, table):
    n_b, s = x.shape
    vocab, d = table.shape
    assert d == D_MODEL and s == S_LEN
    xpad = jnp.pad(x.astype(jnp.int32), ((0, 0), (0, S_PAD - S_LEN)))
    tpad = jnp.pad(table, ((0, 0), (0, D_PAD - D_MODEL)))
    return _build_gather(n_b, vocab)(xpad, tpad)
